# baseline (speedup 1.0000x reference)
import functools

import jax
import jax.numpy as jnp
from jax.experimental import pallas as pl
from jax.experimental.pallas import tpu as pltpu

N = 10000
BLOCK_ROWS = 200


def _gcn_kernel(x_ref, w_ref, adj_ref, bias_ref, out_ref, support_ref):
    @pl.when(pl.program_id(0) == 0)
    def _compute_support():
        support_ref[...] = jnp.dot(
            x_ref[...], w_ref[...], preferred_element_type=jnp.float32
        ).astype(jnp.bfloat16)

    out_ref[...] = (
        jnp.dot(
            adj_ref[...].astype(jnp.bfloat16),
            support_ref[...],
            preferred_element_type=jnp.float32,
        )
        + bias_ref[...]
    )


@functools.partial(jax.jit, static_argnames=())
def kernel(input, adj, weight, bias):
    n, in_f = input.shape
    out_f = weight.shape[1]
    grid = (n // BLOCK_ROWS,)
    return pl.pallas_call(
        _gcn_kernel,
        grid=grid,
        in_specs=[
            pl.BlockSpec((n, in_f), lambda i: (0, 0)),
            pl.BlockSpec((in_f, out_f), lambda i: (0, 0)),
            pl.BlockSpec((BLOCK_ROWS, n), lambda i: (i, 0)),
            pl.BlockSpec((1, out_f), lambda i: (0, 0)),
        ],
        out_specs=pl.BlockSpec((BLOCK_ROWS, out_f), lambda i: (i, 0)),
        out_shape=jax.ShapeDtypeStruct((n, out_f), jnp.float32),
        scratch_shapes=[pltpu.VMEM((n, out_f), jnp.bfloat16)],
        compiler_params=pltpu.CompilerParams(
            dimension_semantics=("arbitrary",),
        ),
    )(input, weight, adj, bias.reshape(1, out_f))


# speedup vs baseline: 1.0167x; 1.0167x over previous
import functools

import jax
import jax.numpy as jnp
from jax.experimental import pallas as pl
from jax.experimental.pallas import tpu as pltpu

N = 10000
BLOCK_ROWS = 400


def _gcn_kernel(x_ref, w_ref, adj_ref, bias_ref, out_ref, support_ref):
    @pl.when(pl.program_id(0) == 0)
    def _compute_support():
        support_ref[...] = jnp.dot(
            x_ref[...], w_ref[...], preferred_element_type=jnp.float32
        ).astype(jnp.bfloat16)

    out_ref[...] = (
        jnp.dot(
            adj_ref[...].astype(jnp.bfloat16),
            support_ref[...],
            preferred_element_type=jnp.float32,
        )
        + bias_ref[...]
    )


@functools.partial(jax.jit, static_argnames=())
def kernel(input, adj, weight, bias):
    n, in_f = input.shape
    out_f = weight.shape[1]
    grid = (n // BLOCK_ROWS,)
    return pl.pallas_call(
        _gcn_kernel,
        grid=grid,
        in_specs=[
            pl.BlockSpec((n, in_f), lambda i: (0, 0)),
            pl.BlockSpec((in_f, out_f), lambda i: (0, 0)),
            pl.BlockSpec((BLOCK_ROWS, n), lambda i: (i, 0)),
            pl.BlockSpec((1, out_f), lambda i: (0, 0)),
        ],
        out_specs=pl.BlockSpec((BLOCK_ROWS, out_f), lambda i: (i, 0)),
        out_shape=jax.ShapeDtypeStruct((n, out_f), jnp.float32),
        scratch_shapes=[pltpu.VMEM((n, out_f), jnp.bfloat16)],
        compiler_params=pltpu.CompilerParams(
            dimension_semantics=("arbitrary",),
        ),
    )(input, weight, adj, bias.reshape(1, out_f))


# speedup vs baseline: 1.0203x; 1.0036x over previous
import functools

import jax
import jax.numpy as jnp
from jax.experimental import pallas as pl
from jax.experimental.pallas import tpu as pltpu

N = 10000
BLOCK_ROWS = 400


NSPLIT = 2


def _gcn_kernel(x_ref, w_ref, *rest):
    adj_refs = rest[:NSPLIT]
    bias_ref = rest[NSPLIT]
    out_ref = rest[NSPLIT + 1]
    support_ref = rest[NSPLIT + 2]

    @pl.when(pl.program_id(0) == 0)
    def _compute_support():
        support_ref[...] = jnp.dot(
            x_ref[...], w_ref[...], preferred_element_type=jnp.float32
        ).astype(jnp.bfloat16)

    sub = BLOCK_ROWS // NSPLIT
    for s in range(NSPLIT):
        out_ref[s * sub : (s + 1) * sub, :] = (
            jnp.dot(
                adj_refs[s][...].astype(jnp.bfloat16),
                support_ref[...],
                preferred_element_type=jnp.float32,
            )
            + bias_ref[...]
        )


@functools.partial(jax.jit, static_argnames=())
def kernel(input, adj, weight, bias):
    n, in_f = input.shape
    out_f = weight.shape[1]
    grid = (n // BLOCK_ROWS,)
    return pl.pallas_call(
        _gcn_kernel,
        grid=grid,
        in_specs=[
            pl.BlockSpec((n, in_f), lambda i: (0, 0)),
            pl.BlockSpec((in_f, out_f), lambda i: (0, 0)),
        ]
        + [
            pl.BlockSpec(
                (BLOCK_ROWS // NSPLIT, n),
                functools.partial(lambda s, i: (i * NSPLIT + s, 0), s),
            )
            for s in range(NSPLIT)
        ]
        + [
            pl.BlockSpec((1, out_f), lambda i: (0, 0)),
        ],
        out_specs=pl.BlockSpec((BLOCK_ROWS, out_f), lambda i: (i, 0)),
        out_shape=jax.ShapeDtypeStruct((n, out_f), jnp.float32),
        scratch_shapes=[pltpu.VMEM((n, out_f), jnp.bfloat16)],
        compiler_params=pltpu.CompilerParams(
            dimension_semantics=("arbitrary",),
        ),
    )(input, weight, *([adj] * NSPLIT), bias.reshape(1, out_f))
